# baseline (speedup 1.0000x reference)
import functools

import jax
import jax.numpy as jnp
from jax import lax
from jax.experimental import pallas as pl
from jax.experimental.pallas import tpu as pltpu
from jax.experimental.pallas import tpu_sc as plsc

M, G, DZ = 16, 16384, 128
NT, DY = 8192, 64
B = M * NT

NC, NS = 2, 16
NW = NC * NS
B_PER_W = B // NW
W_PER_ROW = NT // B_PER_W
CH = 128
NCHUNK = B_PER_W // CH
NBUF = 2


def _sc_gather():
    mesh = plsc.VectorSubcoreMesh(core_axis_name="c", subcore_axis_name="s")

    @functools.partial(
        pl.kernel,
        mesh=mesh,
        out_type=jax.ShapeDtypeStruct((M, NT, DZ), jnp.float32),
        scratch_types=[
            pltpu.VMEM((B_PER_W,), jnp.int32),
            *[pltpu.VMEM((CH, DZ), jnp.float32) for _ in range(NBUF)],
            *[pltpu.SemaphoreType.DMA for _ in range(2 * NBUF)],
        ],
    )
    def gather(table_hbm, mt_hbm, out_hbm, idx_v, *bufs_and_sems):
        rows = bufs_and_sems[:NBUF]
        gsem = bufs_and_sems[NBUF : 2 * NBUF]
        ssem = bufs_and_sems[2 * NBUF :]
        wid = lax.axis_index("s") * NC + lax.axis_index("c")
        i = wid // W_PER_ROW
        h = wid % W_PER_ROW
        col0 = h * B_PER_W
        pltpu.sync_copy(mt_hbm.at[i, pl.ds(col0, B_PER_W)], idx_v)

        def gather_chunk(c, b):
            return pltpu.async_copy(
                table_hbm.at[i].at[idx_v.at[pl.ds(c * CH, CH)]], rows[b], gsem[b]
            )

        gcp = [None] * NBUF
        scp = [None] * NBUF
        gcp[0] = gather_chunk(0, 0)
        for c in range(NCHUNK):
            b = c % NBUF
            nb = (c + 1) % NBUF
            if c + 1 < NCHUNK:
                if scp[nb] is not None:
                    scp[nb].wait()
                gcp[nb] = gather_chunk(c + 1, nb)
            gcp[b].wait()
            scp[b] = pltpu.async_copy(
                rows[b], out_hbm.at[i, pl.ds(col0 + c * CH, CH)], ssem[b]
            )
        for b in range(NBUF):
            if scp[b] is not None:
                scp[b].wait()

    return gather


_gather_fn = _sc_gather()


def _mm_body(zt_ref, w_ref, b_ref, o_ref):
    o_ref[0] = (
        jnp.dot(zt_ref[0], w_ref[...], preferred_element_type=jnp.float32)
        + b_ref[...]
    )


def _tc_linear(zt, W, b2):
    BN = 2048
    return pl.pallas_call(
        _mm_body,
        grid=(M, NT // BN),
        in_specs=[
            pl.BlockSpec((1, BN, DZ), lambda i, j: (i, j, 0)),
            pl.BlockSpec((DZ, DY), lambda i, j: (0, 0)),
            pl.BlockSpec((1, DY), lambda i, j: (0, 0)),
        ],
        out_specs=pl.BlockSpec((1, BN, DY), lambda i, j: (i, j, 0)),
        out_shape=jax.ShapeDtypeStruct((M, NT, DY), jnp.float32),
    )(zt, W, b2)


@jax.jit
def kernel(z_grid, mt, W, b):
    zt = _gather_fn(z_grid, mt.astype(jnp.int32))
    return _tc_linear(zt, W, b.reshape(1, DY))
